# baseline (speedup 1.0000x reference)
import functools

import jax
import jax.numpy as jnp
from jax import lax
from jax.experimental import pallas as pl
from jax.experimental.pallas import tpu as pltpu
from jax.experimental.pallas import tpu_sc as plsc

N_NODES = 10000
E = 320000
D = 128
NW = 32
EW = E // NW
C = 80
NCHUNK = EW // C
NG = C // 16
INV_SQRT_D = 1.0 / (D ** 0.5)


def _sc_body(g_hbm, k_hbm, q_hbm, v_hbm, src_hbm, dst_hbm, out_hbm,
             src_v, dst_v, gs_v, gd_v, ks_v, qd_v, vs_v, o_v, sem):
    wid = lax.axis_index("s") * 2 + lax.axis_index("c")
    lanes = lax.iota(jnp.int32, 16)

    def chunk(i, carry):
        base = wid * EW + i * C
        pltpu.sync_copy(src_hbm.at[pl.ds(base, C)], src_v)
        pltpu.sync_copy(dst_hbm.at[pl.ds(base, C)], dst_v)
        pltpu.async_copy(g_hbm.at[src_v], gs_v, sem).wait()
        pltpu.async_copy(g_hbm.at[dst_v], gd_v, sem).wait()
        pltpu.async_copy(k_hbm.at[src_v], ks_v, sem).wait()
        pltpu.async_copy(q_hbm.at[dst_v], qd_v, sem).wait()
        pltpu.async_copy(v_hbm.at[src_v], vs_v, sem).wait()

        for g in range(NG):
            ew = lanes + (g * 16)

            def feat(f, acc):
                acc_d, acc_s = acc
                fv = jnp.full((16,), f, jnp.int32)
                a = plsc.load_gather(gs_v, [ew, fv])
                b = plsc.load_gather(gd_v, [ew, fv])
                kk = plsc.load_gather(ks_v, [ew, fv])
                qq = plsc.load_gather(qd_v, [ew, fv])
                d = a - b
                return acc_d + d * d, acc_s + kk * qq

            zero = jnp.zeros((16,), jnp.float32)
            acc_d, acc_s = lax.fori_loop(0, D, feat, (zero, zero))

            x = acc_d + 1e-6
            ibits = lax.bitcast_convert_type(x, jnp.int32)
            ibits = 0x5F3759DF - lax.shift_right_logical(ibits, 1)
            y = lax.bitcast_convert_type(ibits, jnp.float32)
            y = y * (1.5 - 0.5 * x * y * y)
            y = y * (1.5 - 0.5 * x * y * y)
            y = y * (1.5 - 0.5 * x * y * y)
            sq = x * y

            dist = jnp.clip(-sq * INV_SQRT_D, -5.0, 5.0)
            score = jnp.clip(acc_s * INV_SQRT_D, -5.0, 5.0)
            w = jnp.exp(dist) * jnp.exp(score)

            def scale(f, c):
                fv = jnp.full((16,), f, jnp.int32)
                v = plsc.load_gather(vs_v, [ew, fv])
                plsc.store_scatter(o_v, [ew, fv], w * v)
                return c

            lax.fori_loop(0, D, scale, 0)

        pltpu.sync_copy(o_v, out_hbm.at[pl.ds(base, C)])
        return carry

    lax.fori_loop(0, NCHUNK, chunk, 0)


@jax.jit
def _run(g, k, q, v, src, dst):
    mesh = plsc.VectorSubcoreMesh(core_axis_name="c", subcore_axis_name="s")
    f = pl.kernel(
        _sc_body,
        mesh=mesh,
        out_type=jax.ShapeDtypeStruct((E, D), jnp.float32),
        scratch_types=[
            pltpu.VMEM((C,), jnp.int32),
            pltpu.VMEM((C,), jnp.int32),
            pltpu.VMEM((C, D), jnp.float32),
            pltpu.VMEM((C, D), jnp.float32),
            pltpu.VMEM((C, D), jnp.float32),
            pltpu.VMEM((C, D), jnp.float32),
            pltpu.VMEM((C, D), jnp.float32),
            pltpu.VMEM((C, D), jnp.float32),
            pltpu.SemaphoreType.DMA,
        ],
        compiler_params=pltpu.CompilerParams(needs_layout_passes=False),
    )
    return f(g, k, q, v, src, dst)


def kernel(G_h, K_h, Q_h, V_h, edge_index):
    src = edge_index[0].astype(jnp.int32)
    dst = edge_index[1].astype(jnp.int32)
    v2d = V_h.reshape(N_NODES, D)
    out = _run(G_h, K_h, Q_h, v2d, src, dst)
    return out.reshape(E, 8, 16)


# speedup vs baseline: 1.7961x; 1.7961x over previous
import jax
import jax.numpy as jnp
from jax import lax
from jax.experimental import pallas as pl
from jax.experimental.pallas import tpu as pltpu
from jax.experimental.pallas import tpu_sc as plsc

N_NODES = 10000
E = 320000
D = 128
NW = 32
EW = E // NW
C = 80
NCHUNK = EW // C
NG = C // 16
SD = 3 * D
DD = 2 * D
UNROLL = 4
INV_SQRT_D = 1.0 / (D ** 0.5)


def _sc_body(srct_hbm, dstt_hbm, src_hbm, dst_hbm, out_hbm,
             src_v, dst_v, sb0, db0, sb1, db1, wb,
             sem0, sem1, semi):
    wid = lax.axis_index("s") * 2 + lax.axis_index("c")
    lanes = lax.iota(jnp.int32, 16)
    wbase = wid * EW

    pltpu.async_copy(src_hbm.at[pl.ds(wbase, EW)], src_v, semi)
    pltpu.async_copy(dst_hbm.at[pl.ds(wbase, EW)], dst_v, semi).wait()
    pltpu.make_async_copy(src_hbm.at[pl.ds(wbase, EW)], src_v, semi).wait()

    bufs = ((sb0, db0, sem0), (sb1, db1, sem1))

    def issue(i, b):
        sb, db, sem = bufs[b]
        pltpu.async_copy(srct_hbm.at[src_v.at[pl.ds(i * C, C)]], sb, sem)
        pltpu.async_copy(dstt_hbm.at[dst_v.at[pl.ds(i * C, C)]], db, sem)

    def wait(i, b):
        sb, db, sem = bufs[b]
        pltpu.make_async_copy(srct_hbm.at[src_v.at[pl.ds(i * C, C)]], sb,
                              sem).wait()
        pltpu.make_async_copy(dstt_hbm.at[dst_v.at[pl.ds(i * C, C)]], db,
                              sem).wait()

    def compute_store(i, b):
        sb, db, sem = bufs[b]
        for g in range(NG):
            ew = lanes + (g * 16)

            def feat(t, acc):
                acc_d, acc_s, fv = acc
                for u in range(UNROLL):
                    fu = fv + u if u else fv
                    fk = jnp.bitwise_or(fu, D)
                    a = plsc.load_gather(sb, [ew, fu])
                    bb = plsc.load_gather(db, [ew, fu])
                    kk = plsc.load_gather(sb, [ew, fk])
                    qq = plsc.load_gather(db, [ew, fk])
                    d = a - bb
                    acc_d = acc_d + d * d
                    acc_s = acc_s + kk * qq
                return acc_d, acc_s, fv + UNROLL

            zero = jnp.zeros((16,), jnp.float32)
            fv0 = jnp.zeros((16,), jnp.int32)
            acc_d, acc_s, _ = lax.fori_loop(0, D // UNROLL, feat,
                                            (zero, zero, fv0))

            x = acc_d + 1e-6
            ibits = lax.bitcast_convert_type(x, jnp.int32)
            ibits = 0x5F3759DF - lax.shift_right_logical(ibits, 1)
            y = lax.bitcast_convert_type(ibits, jnp.float32)
            y = y * (1.5 - 0.5 * x * y * y)
            y = y * (1.5 - 0.5 * x * y * y)
            y = y * (1.5 - 0.5 * x * y * y)
            sq = x * y

            dist = jnp.clip(-sq * INV_SQRT_D, -5.0, 5.0)
            score = jnp.clip(acc_s * INV_SQRT_D, -5.0, 5.0)
            w = jnp.exp(dist) * jnp.exp(score)
            wb[...] = w

            def edge(e, c):
                splat = plsc.load_gather(wb, [jnp.full((16,), e, jnp.int32)])
                re = g * 16 + e
                for j in range(8):
                    col = 2 * D + 16 * j
                    v = sb[re, pl.ds(col, 16)]
                    sb[re, pl.ds(col, 16)] = splat * v
                return c

            lax.fori_loop(0, 16, edge, 0)

        pltpu.sync_copy(sb.at[:, pl.ds(2 * D, D)],
                        out_hbm.at[pl.ds(wbase + i * C, C)])

    issue(0, 0)

    def pair(p, carry):
        i0 = p * 2
        issue(i0 + 1, 1)
        wait(i0, 0)
        compute_store(i0, 0)

        @pl.when(i0 + 2 < NCHUNK)
        def _():
            issue(i0 + 2, 0)

        wait(i0 + 1, 1)
        compute_store(i0 + 1, 1)
        return carry

    lax.fori_loop(0, NCHUNK // 2, pair, 0)
    wait(NCHUNK - 1, 0)
    compute_store(NCHUNK - 1, 0)


@jax.jit
def _run(srct, dstt, src, dst):
    mesh = plsc.VectorSubcoreMesh(core_axis_name="c", subcore_axis_name="s")
    f = pl.kernel(
        _sc_body,
        mesh=mesh,
        out_type=jax.ShapeDtypeStruct((E, D), jnp.float32),
        scratch_types=[
            pltpu.VMEM((EW,), jnp.int32),
            pltpu.VMEM((EW,), jnp.int32),
            pltpu.VMEM((C, SD), jnp.float32),
            pltpu.VMEM((C, DD), jnp.float32),
            pltpu.VMEM((C, SD), jnp.float32),
            pltpu.VMEM((C, DD), jnp.float32),
            pltpu.VMEM((16,), jnp.float32),
            pltpu.SemaphoreType.DMA,
            pltpu.SemaphoreType.DMA,
            pltpu.SemaphoreType.DMA,
        ],
        compiler_params=pltpu.CompilerParams(needs_layout_passes=False),
    )
    return f(srct, dstt, src, dst)


def kernel(G_h, K_h, Q_h, V_h, edge_index):
    src = edge_index[0].astype(jnp.int32)
    dst = edge_index[1].astype(jnp.int32)
    srct = jnp.concatenate([G_h, K_h, V_h.reshape(N_NODES, D)], axis=1)
    dstt = jnp.concatenate([G_h, Q_h], axis=1)
    out = _run(srct, dstt, src, dst)
    return out.reshape(E, 8, 16)


# speedup vs baseline: 3.0214x; 1.6822x over previous
import jax
import jax.numpy as jnp
from jax import lax
from jax.experimental import pallas as pl
from jax.experimental.pallas import tpu as pltpu
from jax.experimental.pallas import tpu_sc as plsc

N_NODES = 10000
E = 320000
D = 128
PD = D // 2
NW = 32
EW = E // NW
C = 80
NCHUNK = EW // C
NG = C // 16
SD = 2 * PD + D
DD = 2 * PD
UNROLL = 4
INV_SQRT_D = 1.0 / (D ** 0.5)


def _sc_body(srct_hbm, dstt_hbm, src_hbm, dst_hbm, out_hbm,
             src_v, dst_v, sb0, db0, sb1, db1, wb,
             sem0, sem1, semi):
    wid = lax.axis_index("s") * 2 + lax.axis_index("c")
    lanes = lax.iota(jnp.int32, 16)
    wbase = wid * EW

    pltpu.async_copy(src_hbm.at[pl.ds(wbase, EW)], src_v, semi)
    pltpu.async_copy(dst_hbm.at[pl.ds(wbase, EW)], dst_v, semi).wait()
    pltpu.make_async_copy(src_hbm.at[pl.ds(wbase, EW)], src_v, semi).wait()

    bufs = ((sb0, db0, sem0), (sb1, db1, sem1))

    def issue(i, b):
        sb, db, sem = bufs[b]
        pltpu.async_copy(srct_hbm.at[src_v.at[pl.ds(i * C, C)]], sb, sem)
        pltpu.async_copy(dstt_hbm.at[dst_v.at[pl.ds(i * C, C)]], db, sem)

    def wait(i, b):
        sb, db, sem = bufs[b]
        pltpu.make_async_copy(srct_hbm.at[src_v.at[pl.ds(i * C, C)]], sb,
                              sem).wait()
        pltpu.make_async_copy(dstt_hbm.at[dst_v.at[pl.ds(i * C, C)]], db,
                              sem).wait()

    def unpk(word):
        return plsc.unpack(plsc.bitcast(word, jnp.bfloat16),
                           format=plsc.PackFormat.INTERLEAVED)

    def compute_store(i, b):
        sb, db, sem = bufs[b]
        for g in range(NG):
            ew = lanes + (g * 16)

            def feat(t, acc):
                acc_d, acc_s, fv = acc
                for u in range(UNROLL):
                    fu = fv + u if u else fv
                    fk = jnp.bitwise_or(fu, PD)
                    gs0, gs1 = unpk(plsc.load_gather(sb, [ew, fu]))
                    gd0, gd1 = unpk(plsc.load_gather(db, [ew, fu]))
                    ks0, ks1 = unpk(plsc.load_gather(sb, [ew, fk]))
                    qd0, qd1 = unpk(plsc.load_gather(db, [ew, fk]))
                    d0 = gs0 - gd0
                    d1 = gs1 - gd1
                    acc_d = acc_d + d0 * d0 + d1 * d1
                    acc_s = acc_s + ks0 * qd0 + ks1 * qd1
                return acc_d, acc_s, fv + UNROLL

            zero = jnp.zeros((16,), jnp.float32)
            fv0 = jnp.zeros((16,), jnp.int32)
            acc_d, acc_s, _ = lax.fori_loop(0, PD // UNROLL, feat,
                                            (zero, zero, fv0))

            x = acc_d + 1e-6
            ibits = lax.bitcast_convert_type(x, jnp.int32)
            ibits = 0x5F3759DF - lax.shift_right_logical(ibits, 1)
            y = lax.bitcast_convert_type(ibits, jnp.float32)
            y = y * (1.5 - 0.5 * x * y * y)
            y = y * (1.5 - 0.5 * x * y * y)
            y = y * (1.5 - 0.5 * x * y * y)
            sq = x * y

            dist = jnp.clip(-sq * INV_SQRT_D, -5.0, 5.0)
            score = jnp.clip(acc_s * INV_SQRT_D, -5.0, 5.0)
            w = jnp.exp(dist) * jnp.exp(score)
            wb[...] = w

            def edge(e, c):
                splat = plsc.load_gather(wb, [jnp.full((16,), e, jnp.int32)])
                re = g * 16 + e
                for j in range(8):
                    col = 2 * PD + 16 * j
                    v = sb[re, pl.ds(col, 16)]
                    sb[re, pl.ds(col, 16)] = splat * v
                return c

            lax.fori_loop(0, 16, edge, 0)

        pltpu.sync_copy(sb.at[:, pl.ds(2 * PD, D)],
                        out_hbm.at[pl.ds(wbase + i * C, C)])

    issue(0, 0)

    def pair(p, carry):
        i0 = p * 2
        issue(i0 + 1, 1)
        wait(i0, 0)
        compute_store(i0, 0)

        @pl.when(i0 + 2 < NCHUNK)
        def _():
            issue(i0 + 2, 0)

        wait(i0 + 1, 1)
        compute_store(i0 + 1, 1)
        return carry

    lax.fori_loop(0, NCHUNK // 2, pair, 0)
    wait(NCHUNK - 1, 0)
    compute_store(NCHUNK - 1, 0)


@jax.jit
def _run(srct, dstt, src, dst):
    mesh = plsc.VectorSubcoreMesh(core_axis_name="c", subcore_axis_name="s")
    f = pl.kernel(
        _sc_body,
        mesh=mesh,
        out_type=jax.ShapeDtypeStruct((E, D), jnp.float32),
        scratch_types=[
            pltpu.VMEM((EW,), jnp.int32),
            pltpu.VMEM((EW,), jnp.int32),
            pltpu.VMEM((C, SD), jnp.float32),
            pltpu.VMEM((C, DD), jnp.float32),
            pltpu.VMEM((C, SD), jnp.float32),
            pltpu.VMEM((C, DD), jnp.float32),
            pltpu.VMEM((16,), jnp.float32),
            pltpu.SemaphoreType.DMA,
            pltpu.SemaphoreType.DMA,
            pltpu.SemaphoreType.DMA,
        ],
        compiler_params=pltpu.CompilerParams(needs_layout_passes=False),
    )
    return f(srct, dstt, src, dst)


def _pack_bf16(x):
    xb = x.astype(jnp.bfloat16).reshape(N_NODES, PD, 2)
    return lax.bitcast_convert_type(xb, jnp.float32)


def kernel(G_h, K_h, Q_h, V_h, edge_index):
    src = edge_index[0].astype(jnp.int32)
    dst = edge_index[1].astype(jnp.int32)
    gp = _pack_bf16(G_h)
    kp = _pack_bf16(K_h)
    qp = _pack_bf16(Q_h)
    srct = jnp.concatenate([gp, kp, V_h.reshape(N_NODES, D)], axis=1)
    dstt = jnp.concatenate([gp, qp], axis=1)
    out = _run(srct, dstt, src, dst)
    return out.reshape(E, 8, 16)
